# baseline (speedup 1.0000x reference)
import jax
import jax.numpy as jnp
from jax import lax
from jax.experimental import pallas as pl
from jax.experimental.pallas import tpu as pltpu
from jax.experimental.pallas import tpu_sc as plsc

VOCAB = 1000000
DIM = 32
BATCH = 16384

NC = 2
NS = 16
L = 16
NW = NC * NS
B_PER_W = BATCH // NW
IDX_CHUNK = 128
N_CHUNKS = B_PER_W // IDX_CHUNK
N_GROUPS = B_PER_W // L
FLAT = DIM * VOCAB


def _sc_body(i_hbm, j_hbm, wi_hbm, wj_hbm, bi_hbm, bj_hbm, out_hbm,
             idx_i, idx_j, pix_i, pix_j, rows_i, rows_j, br_i, br_j,
             out_v, sem, bsem):
    wid = lax.axis_index("s") * NC + lax.axis_index("c")
    base = wid * B_PER_W

    pltpu.sync_copy(i_hbm.at[wid], idx_i)
    pltpu.sync_copy(j_hbm.at[wid], idx_j)

    bias_copies = []
    for c in range(N_CHUNKS):
        sl = pl.ds(c * IDX_CHUNK, IDX_CHUNK)
        bias_copies.append(
            pltpu.async_copy(bi_hbm.at[idx_i.at[c]], br_i.at[sl], bsem))
        bias_copies.append(
            pltpu.async_copy(bj_hbm.at[idx_i.at[c]], br_j.at[sl], bsem))

    def pix_body(k, carry):
        c = k // (IDX_CHUNK // L)
        s = pl.ds((k % (IDX_CHUNK // L)) * L, L)
        pvi = idx_i[c, s] << 5
        pvj = idx_j[c, s] << 5
        for d in range(DIM):
            pix_i[d, c, s] = pvi + d
            pix_j[d, c, s] = pvj + d
        return carry

    lax.fori_loop(0, N_CHUNKS * (IDX_CHUNK // L), pix_body, 0)

    def fire(d):
        for c in range(N_CHUNKS):
            sl = pl.ds(c * IDX_CHUNK, IDX_CHUNK)
            pltpu.async_copy(wi_hbm.at[pix_i.at[d, c]], rows_i.at[d, sl], sem)
            pltpu.async_copy(wj_hbm.at[pix_j.at[d, c]], rows_j.at[d, sl], sem)

    def drain(d):
        for c in range(N_CHUNKS):
            sl = pl.ds(c * IDX_CHUNK, IDX_CHUNK)
            pltpu.make_async_copy(
                wi_hbm.at[pix_i.at[d, c]], rows_i.at[d, sl], sem).wait()
            pltpu.make_async_copy(
                wj_hbm.at[pix_j.at[d, c]], rows_j.at[d, sl], sem).wait()

    def fire_body(d, carry):
        @pl.when(d > 0)
        def _():
            drain(d - 1)
        fire(d)
        return carry

    lax.fori_loop(0, DIM, fire_body, 0)
    drain(DIM - 1)
    for cp in bias_copies:
        cp.wait()

    def group_body(g, carry):
        s = pl.ds(g * L, L)
        acc = br_i[s] + br_j[s]
        for d in range(DIM):
            acc = acc + rows_i[d, s] * rows_j[d, s]
        out_v[s] = acc
        return carry

    lax.fori_loop(0, N_GROUPS, group_body, 0)

    pltpu.sync_copy(out_v, out_hbm.at[pl.ds(base, B_PER_W)])


@jax.jit
def _run(i2, j2, wi_f, wj_f, bi_f, bj_f):
    mesh = plsc.VectorSubcoreMesh(
        core_axis_name="c", subcore_axis_name="s",
        num_cores=NC, num_subcores=NS)
    return pl.kernel(
        _sc_body,
        out_type=jax.ShapeDtypeStruct((BATCH,), jnp.float32),
        mesh=mesh,
        compiler_params=pltpu.CompilerParams(
            needs_layout_passes=False, use_tc_tiling_on_sc=False),
        scratch_types=[
            pltpu.VMEM((N_CHUNKS, IDX_CHUNK), jnp.int32),
            pltpu.VMEM((N_CHUNKS, IDX_CHUNK), jnp.int32),
            pltpu.VMEM((DIM, N_CHUNKS, IDX_CHUNK), jnp.int32),
            pltpu.VMEM((DIM, N_CHUNKS, IDX_CHUNK), jnp.int32),
            pltpu.VMEM((DIM, B_PER_W), jnp.float32),
            pltpu.VMEM((DIM, B_PER_W), jnp.float32),
            pltpu.VMEM((B_PER_W,), jnp.float32),
            pltpu.VMEM((B_PER_W,), jnp.float32),
            pltpu.VMEM((B_PER_W,), jnp.float32),
            pltpu.SemaphoreType.DMA,
            pltpu.SemaphoreType.DMA,
        ],
    )(i2, j2, wi_f, wj_f, bi_f, bj_f)


def kernel(i, j, wi, wj, bi, bj):
    i2 = i.reshape(NW, N_CHUNKS, IDX_CHUNK)
    j2 = j.reshape(NW, N_CHUNKS, IDX_CHUNK)
    return _run(i2, j2, wi.reshape(FLAT), wj.reshape(FLAT),
                bi.reshape(VOCAB), bj.reshape(VOCAB))


# speedup vs baseline: 1.0507x; 1.0507x over previous
import jax
import jax.numpy as jnp
from jax import lax
from jax.experimental import pallas as pl
from jax.experimental.pallas import tpu as pltpu
from jax.experimental.pallas import tpu_sc as plsc

VOCAB = 1000000
DIM = 32
BATCH = 16384

NC = 2
NS = 16
L = 16
NW = NC * NS
B_PER_W = BATCH // NW
IDX_CHUNK = 128
N_CHUNKS = B_PER_W // IDX_CHUNK
N_GROUPS = B_PER_W // L


def _sc_body(i_hbm, j_hbm, wi_hbm, wj_hbm, bi_hbm, bj_hbm, out_hbm,
             idx_i, idx_j, rows_i, rows_j, br_i, br_j, out_v, sem):
    wid = lax.axis_index("s") * NC + lax.axis_index("c")
    base = wid * B_PER_W

    pltpu.sync_copy(i_hbm.at[wid], idx_i)
    pltpu.sync_copy(j_hbm.at[wid], idx_j)

    copies = []
    for c in range(N_CHUNKS):
        sl = pl.ds(c * IDX_CHUNK, IDX_CHUNK)
        copies.append(pltpu.async_copy(wi_hbm.at[idx_i.at[c]], rows_i.at[sl], sem))
        copies.append(pltpu.async_copy(wj_hbm.at[idx_j.at[c]], rows_j.at[sl], sem))
        copies.append(pltpu.async_copy(bi_hbm.at[idx_i.at[c]], br_i.at[sl], sem))
        copies.append(pltpu.async_copy(bj_hbm.at[idx_i.at[c]], br_j.at[sl], sem))
    for cp in copies:
        cp.wait()

    lanes = lax.iota(jnp.int32, L)

    def group_body(g, carry):
        row_ids = g * L + lanes
        acc = plsc.load_gather(br_i, [row_ids])
        acc = acc + plsc.load_gather(br_j, [row_ids])
        for d in range(DIM):
            dcol = jnp.full((L,), d, jnp.int32)
            vi = plsc.load_gather(rows_i, [row_ids, dcol])
            vj = plsc.load_gather(rows_j, [row_ids, dcol])
            acc = acc + vi * vj
        out_v[pl.ds(g * L, L)] = acc
        return carry

    lax.fori_loop(0, N_GROUPS, group_body, 0)

    pltpu.sync_copy(out_v, out_hbm.at[pl.ds(base, B_PER_W)])


@jax.jit
def _run(i2, j2, wi, wj, bi, bj):
    mesh = plsc.VectorSubcoreMesh(
        core_axis_name="c", subcore_axis_name="s",
        num_cores=NC, num_subcores=NS)
    return pl.kernel(
        _sc_body,
        out_type=jax.ShapeDtypeStruct((BATCH,), jnp.float32),
        mesh=mesh,
        compiler_params=pltpu.CompilerParams(
            needs_layout_passes=False, use_tc_tiling_on_sc=False),
        scratch_types=[
            pltpu.VMEM((N_CHUNKS, IDX_CHUNK), jnp.int32),
            pltpu.VMEM((N_CHUNKS, IDX_CHUNK), jnp.int32),
            pltpu.VMEM((B_PER_W, DIM), jnp.float32),
            pltpu.VMEM((B_PER_W, DIM), jnp.float32),
            pltpu.VMEM((B_PER_W,), jnp.float32),
            pltpu.VMEM((B_PER_W,), jnp.float32),
            pltpu.VMEM((B_PER_W,), jnp.float32),
            pltpu.SemaphoreType.DMA,
        ],
    )(i2, j2, wi, wj, bi, bj)


def kernel(i, j, wi, wj, bi, bj):
    i2 = i.reshape(NW, N_CHUNKS, IDX_CHUNK)
    j2 = j.reshape(NW, N_CHUNKS, IDX_CHUNK)
    return _run(i2, j2, wi, wj, bi.reshape(VOCAB), bj.reshape(VOCAB))


# speedup vs baseline: 2.9232x; 2.7820x over previous
import jax
import jax.numpy as jnp
from jax import lax
from jax.experimental import pallas as pl
from jax.experimental.pallas import tpu as pltpu
from jax.experimental.pallas import tpu_sc as plsc

VOCAB = 1000000
DIM = 32
BATCH = 16384

NC = 2
NS = 16
L = 16
NW = NC * NS
B_PER_W = BATCH // NW
IDX_CHUNK = 128
N_CHUNKS = B_PER_W // IDX_CHUNK
N_GROUPS = B_PER_W // L

NB = 7816
VPAD = NB * 128
NGRP = NB // 8
NGFULL = NGRP - 1
GTB = NGFULL * 8
GPW = NGFULL // NW
NGREM = NGFULL - GPW * NW


def _detile_body(wi3, wj3, owi, owj, buf, rsem, wsem):
    wid = lax.axis_index("s") * NC + lax.axis_index("c")
    g0 = wid * GPW
    n = GPW

    def fire_reads(src3, g, s):
        for t in range(4):
            for blk in range(8):
                off = (g * 8 + blk) * 128
                pltpu.async_copy(src3.at[t, :, pl.ds(off, 128)],
                                 buf.at[s, t, :, 0, blk, :], rsem)

    def drain_reads(src3, g, s):
        for t in range(4):
            for blk in range(8):
                off = (g * 8 + blk) * 128
                pltpu.make_async_copy(src3.at[t, :, pl.ds(off, 128)],
                                      buf.at[s, t, :, 0, blk, :], rsem).wait()

    def fire_writes(dst5, g, s):
        for t in range(4):
            pltpu.async_copy(buf.at[s, t], dst5.at[t, :, pl.ds(g, 1), :, :], wsem)

    def drain_writes(dst5, g, s):
        for t in range(4):
            pltpu.make_async_copy(buf.at[s, t], dst5.at[t, :, pl.ds(g, 1), :, :], wsem).wait()

    pairs = ((wi3, owi), (wj3, owj))
    for src3, dst5 in pairs:
        def body(k, carry):
            g = g0 + k
            s = k & 1

            @pl.when(k > 1)
            def _():
                drain_writes(dst5, g - 2, s)
            fire_reads(src3, g, s)

            @pl.when(k > 0)
            def _():
                drain_reads(src3, g - 1, s ^ 1)
                fire_writes(dst5, g - 1, s ^ 1)
            return carry

        lax.fori_loop(0, n, body, 0)
        if True:
            last = g0 + n - 1
            ls = (n - 1) & 1
            drain_writes(dst5, last - 1, ls ^ 1)
            drain_reads(src3, last, ls)
            fire_writes(dst5, last, ls)
            drain_writes(dst5, last, ls)

        if True:
            @pl.when(wid < NGREM)
            def _():
                g = NW * GPW + wid
                fire_reads(src3, g, 0)
                drain_reads(src3, g, 0)
                fire_writes(dst5, g, 0)
                drain_writes(dst5, g, 0)

    if True:
        @pl.when(wid == 0)
        def _():
            for src3, dst5 in pairs:
                for blk in range(4):
                    off = (GTB + blk) * 128
                    for t in range(4):
                        pltpu.sync_copy(src3.at[t, :, pl.ds(off, 128)],
                                        buf.at[0, t, :, 0, blk, :])
                for t in range(4):
                    for d8 in range(8):
                        pltpu.sync_copy(
                            src3.at[t, d8, pl.ds(999936, 64)],
                            buf.at[0, t, d8, 0, 4, pl.ds(0, 64)])
                fire_writes(dst5, NGFULL, 0)
                drain_writes(dst5, NGFULL, 0)


def _gather_body(i_hbm, j_hbm, wi_hbm, wj_hbm, bi_hbm, bj_hbm, out_hbm,
                 idx_i, idx_j, rows_i, rows_j, br_i, br_j, out_v, sem, bsem):
    wid = lax.axis_index("s") * NC + lax.axis_index("c")
    base = wid * B_PER_W

    pltpu.sync_copy(i_hbm.at[wid], idx_i)
    pltpu.sync_copy(j_hbm.at[wid], idx_j)

    bias_copies = []
    for c in range(N_CHUNKS):
        sl = pl.ds(c * IDX_CHUNK, IDX_CHUNK)
        bias_copies.append(
            pltpu.async_copy(bi_hbm.at[idx_i.at[c]], br_i.at[sl], bsem))
        bias_copies.append(
            pltpu.async_copy(bj_hbm.at[idx_i.at[c]], br_j.at[sl], bsem))

    def fire(d):
        for c in range(N_CHUNKS):
            sl = pl.ds(c * IDX_CHUNK, IDX_CHUNK)
            pltpu.async_copy(wi_hbm.at[d].at[idx_i.at[c]], rows_i.at[d, sl], sem)
            pltpu.async_copy(wj_hbm.at[d].at[idx_j.at[c]], rows_j.at[d, sl], sem)

    def drain(d):
        for c in range(N_CHUNKS):
            sl = pl.ds(c * IDX_CHUNK, IDX_CHUNK)
            pltpu.make_async_copy(
                wi_hbm.at[d].at[idx_i.at[c]], rows_i.at[d, sl], sem).wait()
            pltpu.make_async_copy(
                wj_hbm.at[d].at[idx_j.at[c]], rows_j.at[d, sl], sem).wait()

    def fire_body(d, carry):
        @pl.when(d > 0)
        def _():
            drain(d - 1)
        fire(d)
        return carry

    lax.fori_loop(0, DIM, fire_body, 0)
    drain(DIM - 1)
    for cp in bias_copies:
        cp.wait()

    def group_body(g, carry):
        s = pl.ds(g * L, L)
        acc = br_i[s] + br_j[s]
        for d in range(DIM):
            acc = acc + rows_i[d, s] * rows_j[d, s]
        out_v[s] = acc
        return carry

    lax.fori_loop(0, N_GROUPS, group_body, 0)

    pltpu.sync_copy(out_v, out_hbm.at[pl.ds(base, B_PER_W)])


@jax.jit
def _run(i2, j2, wi3, wj3, bi_f, bj_f):
    mesh = plsc.VectorSubcoreMesh(
        core_axis_name="c", subcore_axis_name="s",
        num_cores=NC, num_subcores=NS)
    owi, owj = pl.kernel(
        _detile_body,
        out_type=(jax.ShapeDtypeStruct((4, 8, NGRP, 8, 128), jnp.float32),
                  jax.ShapeDtypeStruct((4, 8, NGRP, 8, 128), jnp.float32)),
        mesh=mesh,
        compiler_params=pltpu.CompilerParams(needs_layout_passes=False),
        scratch_types=[
            pltpu.VMEM((2, 4, 8, 1, 8, 128), jnp.float32),
            pltpu.SemaphoreType.DMA,
            pltpu.SemaphoreType.DMA,
        ],
    )(wi3, wj3)
    return pl.kernel(
        _gather_body,
        out_type=jax.ShapeDtypeStruct((BATCH,), jnp.float32),
        mesh=mesh,
        compiler_params=pltpu.CompilerParams(
            needs_layout_passes=False, use_tc_tiling_on_sc=False),
        scratch_types=[
            pltpu.VMEM((N_CHUNKS, IDX_CHUNK), jnp.int32),
            pltpu.VMEM((N_CHUNKS, IDX_CHUNK), jnp.int32),
            pltpu.VMEM((DIM, B_PER_W), jnp.float32),
            pltpu.VMEM((DIM, B_PER_W), jnp.float32),
            pltpu.VMEM((B_PER_W,), jnp.float32),
            pltpu.VMEM((B_PER_W,), jnp.float32),
            pltpu.VMEM((B_PER_W,), jnp.float32),
            pltpu.SemaphoreType.DMA,
            pltpu.SemaphoreType.DMA,
        ],
    )(i2, j2, owi.reshape(DIM, VPAD), owj.reshape(DIM, VPAD), bi_f, bj_f)


def kernel(i, j, wi, wj, bi, bj):
    i2 = i.reshape(NW, N_CHUNKS, IDX_CHUNK)
    j2 = j.reshape(NW, N_CHUNKS, IDX_CHUNK)
    wi3 = wi.T.reshape(4, 8, VOCAB)
    wj3 = wj.T.reshape(4, 8, VOCAB)
    return _run(i2, j2, wi3, wj3, bi.reshape(VOCAB), bj.reshape(VOCAB))
